# baseline (speedup 1.0000x reference)
import jax
import jax.numpy as jnp
from jax.experimental import pallas as pl
from jax.experimental.pallas import tpu as pltpu


def _gcn_block(adj_ref, xb_ref, xm_ref, deg_ref, w_ref, b_ref, out_ref):
    adj_bf = adj_ref[...].astype(jnp.bfloat16)
    support = jnp.dot(adj_bf, xb_ref[...],
                      preferred_element_type=jnp.float32) + xm_ref[...]
    node_linear = jnp.dot(support, w_ref[...],
                          preferred_element_type=jnp.float32)
    out_ref[...] = node_linear / deg_ref[...] + b_ref[...]


def kernel(input, adj, node_degs, weight, bias):
    n, f_in = input.shape
    f_out = weight.shape[1]
    bm = 256
    bias2 = bias.reshape(1, f_out)
    x_bf = input.astype(jnp.bfloat16)
    return pl.pallas_call(
        _gcn_block,
        grid=(n // bm,),
        in_specs=[
            pl.BlockSpec((bm, n), lambda i: (i, 0)),
            pl.BlockSpec((n, f_in), lambda i: (0, 0)),
            pl.BlockSpec((bm, f_in), lambda i: (i, 0)),
            pl.BlockSpec((bm, 1), lambda i: (i, 0)),
            pl.BlockSpec((f_in, f_out), lambda i: (0, 0)),
            pl.BlockSpec((1, f_out), lambda i: (0, 0)),
        ],
        out_specs=pl.BlockSpec((bm, f_out), lambda i: (i, 0)),
        out_shape=jax.ShapeDtypeStruct((n, f_out), jnp.float32),
        compiler_params=pltpu.CompilerParams(
            dimension_semantics=("parallel",),
        ),
    )(adj, x_bf, input, node_degs, weight, bias2)


# speedup vs baseline: 1.1260x; 1.1260x over previous
import jax
import jax.numpy as jnp
from jax.experimental import pallas as pl
from jax.experimental.pallas import tpu as pltpu

_BM = 512


def _gcn_block(adj_ref, x_ref, xm_ref, deg_ref, w_ref, b_ref, out_ref):
    support = jnp.dot(adj_ref[...], x_ref[...],
                      preferred_element_type=jnp.float32) + xm_ref[...]
    node_linear = jnp.dot(support, w_ref[...],
                          preferred_element_type=jnp.float32)
    out_ref[...] = node_linear / deg_ref[...] + b_ref[...]


def kernel(input, adj, node_degs, weight, bias):
    n, f_in = input.shape
    f_out = weight.shape[1]
    bm = _BM
    bias2 = bias.reshape(1, f_out)
    return pl.pallas_call(
        _gcn_block,
        grid=(n // bm,),
        in_specs=[
            pl.BlockSpec((bm, n), lambda i: (i, 0)),
            pl.BlockSpec((n, f_in), lambda i: (0, 0)),
            pl.BlockSpec((bm, f_in), lambda i: (i, 0)),
            pl.BlockSpec((bm, 1), lambda i: (i, 0)),
            pl.BlockSpec((f_in, f_out), lambda i: (0, 0)),
            pl.BlockSpec((1, f_out), lambda i: (0, 0)),
        ],
        out_specs=pl.BlockSpec((bm, f_out), lambda i: (i, 0)),
        out_shape=jax.ShapeDtypeStruct((n, f_out), jnp.float32),
        compiler_params=pltpu.CompilerParams(
            dimension_semantics=("parallel",),
        ),
    )(adj, input, input, node_degs, weight, bias2)
